# baseline (speedup 1.0000x reference)
import functools

import jax
import jax.numpy as jnp
from jax import lax
from jax.experimental import pallas as pl
from jax.experimental.pallas import tpu as pltpu
from jax.experimental.pallas import tpu_sc as plsc

_SC_INFO = plsc.get_sparse_core_info()
_NC = _SC_INFO.num_cores
_NS = _SC_INFO.num_subcores
_NW = _NC * _NS


def kernel(x, table):
    seq_len = x.shape[-1]
    hidden = table.shape[1]
    rows_per_w = seq_len // _NW
    mesh = plsc.VectorSubcoreMesh(core_axis_name="c", subcore_axis_name="s")

    @functools.partial(
        pl.kernel,
        mesh=mesh,
        out_type=jax.ShapeDtypeStruct((seq_len, hidden), table.dtype),
    )
    def _copy(table_hbm, out_hbm):
        wid = lax.axis_index("s") * _NC + lax.axis_index("c")
        base = wid * rows_per_w
        pltpu.sync_copy(
            table_hbm.at[pl.ds(base, rows_per_w)],
            out_hbm.at[pl.ds(base, rows_per_w)],
        )

    return _copy(table)


# speedup vs baseline: 16.1622x; 16.1622x over previous
import functools

import jax
import jax.numpy as jnp
from jax import lax
from jax.experimental import pallas as pl
from jax.experimental.pallas import tpu as pltpu
from jax.experimental.pallas import tpu_sc as plsc

_SC_INFO = plsc.get_sparse_core_info()
_NC = _SC_INFO.num_cores
_NS = _SC_INFO.num_subcores
_NW = _NC * _NS

_CHUNK_ROWS = 32


def kernel(x, table):
    seq_len = x.shape[-1]
    hidden = table.shape[1]
    rows_per_w = seq_len // _NW
    nchunks = rows_per_w // _CHUNK_ROWS
    mesh = plsc.VectorSubcoreMesh(core_axis_name="c", subcore_axis_name="s")

    @functools.partial(
        pl.kernel,
        mesh=mesh,
        out_type=jax.ShapeDtypeStruct((seq_len, hidden), table.dtype),
        scratch_types=[
            pltpu.VMEM((2, _CHUNK_ROWS, hidden), table.dtype),
            pltpu.SemaphoreType.DMA,
            pltpu.SemaphoreType.DMA,
            pltpu.SemaphoreType.DMA,
            pltpu.SemaphoreType.DMA,
        ],
    )
    def _copy(table_hbm, out_hbm, buf, lsem0, lsem1, ssem0, ssem1):
        wid = lax.axis_index("s") * _NC + lax.axis_index("c")
        base = wid * rows_per_w
        lsems = (lsem0, lsem1)
        ssems = (ssem0, ssem1)
        loads = [None, None]
        stores = [None, None]
        loads[0] = pltpu.async_copy(
            table_hbm.at[pl.ds(base, _CHUNK_ROWS)], buf.at[0], lsems[0]
        )
        for i in range(nchunks):
            b = i % 2
            nb = (i + 1) % 2
            loads[b].wait()
            if i + 1 < nchunks:
                if stores[nb] is not None:
                    stores[nb].wait()
                loads[nb] = pltpu.async_copy(
                    table_hbm.at[pl.ds(base + (i + 1) * _CHUNK_ROWS, _CHUNK_ROWS)],
                    buf.at[nb],
                    lsems[nb],
                )
            stores[b] = pltpu.async_copy(
                buf.at[b],
                out_hbm.at[pl.ds(base + i * _CHUNK_ROWS, _CHUNK_ROWS)],
                ssems[b],
            )
        for h in stores:
            if h is not None:
                h.wait()

    return _copy(table)
